# baseline (speedup 1.0000x reference)
import functools

import jax
import jax.numpy as jnp
from jax import lax
from jax.experimental import pallas as pl
from jax.experimental.pallas import tpu as pltpu
from jax.experimental.pallas import tpu_sc as plsc

N = 10000
NPAD = 10240
E = 320000
D = 128
C = 16

NC = 2
NS = 16
NW = NC * NS
EPW = E // NW
CHUNK = 128
NCH = 80
EPWP = NCH * CHUNK
NBUF = 2
NGRP = NCH // NBUF
RPT = NPAD // NS
ZROWS = 128

_mesh = plsc.VectorSubcoreMesh(core_axis_name="c", subcore_axis_name="s")


@functools.partial(
    pl.kernel,
    out_type=jax.ShapeDtypeStruct((NC, NPAD, 16), jnp.float32),
    mesh=_mesh,
    scratch_types=[
        pltpu.VMEM_SHARED((NPAD, 16), jnp.float32),
        pltpu.VMEM((CHUNK,), jnp.int32),
        pltpu.VMEM((CHUNK, 16), jnp.float32),
        pltpu.VMEM((RPT, 16), jnp.float32),
    ],
)
def _deg_kernel(dst_hbm, out_hbm, deg_sh, didx, ones_v, stage):
    cid = lax.axis_index("c")
    sid = lax.axis_index("s")
    wid = sid * NC + cid

    def fill(i, _):
        ones_v[i, pl.ds(0, 16)] = jnp.full((16,), 1.0, jnp.float32)
        return 0
    lax.fori_loop(0, CHUNK, fill, 0)

    def zfill(i, _):
        stage[i, pl.ds(0, 16)] = jnp.zeros((16,), jnp.float32)
        return 0
    lax.fori_loop(0, RPT, zfill, 0)

    r0 = sid * RPT
    pltpu.sync_copy(stage, deg_sh.at[pl.ds(r0, RPT)])
    plsc.subcore_barrier()

    def body(c, _):
        pltpu.sync_copy(dst_hbm.at[wid, c], didx)
        pltpu.sync_copy(ones_v, deg_sh.at[didx], add=True)
        return 0
    lax.fori_loop(0, NCH, body, 0)

    plsc.subcore_barrier()
    pltpu.sync_copy(deg_sh.at[pl.ds(r0, RPT)], stage)
    pltpu.sync_copy(stage, out_hbm.at[cid, pl.ds(r0, RPT)])


@functools.partial(
    pl.kernel,
    out_type=jax.ShapeDtypeStruct((NC, NPAD, D), jnp.float32),
    mesh=_mesh,
    scratch_types=[
        pltpu.VMEM_SHARED((NPAD, D), jnp.float32),
        pltpu.VMEM((CHUNK,), jnp.int32),
        pltpu.VMEM((CHUNK,), jnp.int32),
        pltpu.VMEM((CHUNK,), jnp.int32),
        pltpu.VMEM((CHUNK,), jnp.int32),
        pltpu.VMEM((CHUNK, D), jnp.float32),
        pltpu.VMEM((CHUNK, D), jnp.float32),
        pltpu.SemaphoreType.DMA,
        pltpu.SemaphoreType.DMA,
    ],
)
def _agg_kernel(hs_hbm, src_hbm, dst_hbm, out_hbm, acc_sh,
                six0, dix0, six1, dix1, rb0, rb1, sem0, sem1):
    cid = lax.axis_index("c")
    sid = lax.axis_index("s")
    wid = sid * NC + cid

    def zfill(i, _):
        def zf2(j, _):
            rb0[i, pl.ds(j * 16, 16)] = jnp.zeros((16,), jnp.float32)
            return 0
        lax.fori_loop(0, D // 16, zf2, 0)
        return 0
    lax.fori_loop(0, ZROWS, zfill, 0)

    r0 = sid * RPT
    for k in range(RPT // ZROWS):
        pltpu.sync_copy(rb0, acc_sh.at[pl.ds(r0 + k * ZROWS, ZROWS)])
    plsc.subcore_barrier()

    def body(g, _):
        c = 2 * g
        pltpu.sync_copy(src_hbm.at[wid, c], six0)
        pltpu.sync_copy(dst_hbm.at[wid, c], dix0)
        da = pltpu.async_copy(hs_hbm.at[six0], rb0, sem0)
        pltpu.sync_copy(src_hbm.at[wid, c + 1], six1)
        pltpu.sync_copy(dst_hbm.at[wid, c + 1], dix1)
        db = pltpu.async_copy(hs_hbm.at[six1], rb1, sem1)
        da.wait()
        pltpu.sync_copy(rb0, acc_sh.at[dix0], add=True)
        db.wait()
        pltpu.sync_copy(rb1, acc_sh.at[dix1], add=True)
        return 0

    lax.fori_loop(0, NCH // 2, body, 0)

    plsc.subcore_barrier()
    for k in range(RPT // ZROWS):
        pltpu.sync_copy(acc_sh.at[pl.ds(r0 + k * ZROWS, ZROWS)], rb0)
        pltpu.sync_copy(rb0, out_hbm.at[cid, pl.ds(r0 + k * ZROWS, ZROWS)])


BN = 1024
GRID = (N + BN - 1) // BN


def _mm_body(x_ref, w_ref, o_ref):
    o_ref[...] = jnp.dot(x_ref[...], w_ref[...],
                         preferred_element_type=jnp.float32)


_h1p_call = pl.pallas_call(
    _mm_body,
    grid=(GRID,),
    in_specs=[pl.BlockSpec((BN, D), lambda i: (i, 0)),
              pl.BlockSpec((D, D), lambda i: (0, 0))],
    out_specs=pl.BlockSpec((BN, D), lambda i: (i, 0)),
    out_shape=jax.ShapeDtypeStruct((N, D), jnp.float32),
)


def _tc2_body(h1p_ref, degs_ref, hs1_ref, dis_ref):
    deg = degs_ref[0, :, 0:1] + degs_ref[1, :, 0:1] + 1.0
    dis = lax.rsqrt(deg)
    dis_ref[...] = dis
    hs1_ref[...] = h1p_ref[...] * dis


_tc2_call = pl.pallas_call(
    _tc2_body,
    grid=(GRID,),
    in_specs=[pl.BlockSpec((BN, D), lambda i: (i, 0)),
              pl.BlockSpec((NC, BN, 16), lambda i: (0, i, 0))],
    out_specs=[pl.BlockSpec((BN, D), lambda i: (i, 0)),
               pl.BlockSpec((BN, 1), lambda i: (i, 0))],
    out_shape=[jax.ShapeDtypeStruct((N, D), jnp.float32),
               jax.ShapeDtypeStruct((N, 1), jnp.float32)],
)


def _tc3_body(p_ref, hs1_ref, dis_ref, b1_ref, w2_ref, h1_ref, hs2_ref):
    dis = dis_ref[...]
    agg = p_ref[0] + p_ref[1] + hs1_ref[...]
    h1 = jnp.maximum(dis * agg + b1_ref[...], 0.0)
    h1_ref[...] = h1
    hs2_ref[...] = jnp.dot(h1, w2_ref[...],
                           preferred_element_type=jnp.float32) * dis


_tc3_call = pl.pallas_call(
    _tc3_body,
    grid=(GRID,),
    in_specs=[pl.BlockSpec((NC, BN, D), lambda i: (0, i, 0)),
              pl.BlockSpec((BN, D), lambda i: (i, 0)),
              pl.BlockSpec((BN, 1), lambda i: (i, 0)),
              pl.BlockSpec((1, D), lambda i: (0, 0)),
              pl.BlockSpec((D, D), lambda i: (0, 0))],
    out_specs=[pl.BlockSpec((BN, D), lambda i: (i, 0)),
               pl.BlockSpec((BN, D), lambda i: (i, 0))],
    out_shape=[jax.ShapeDtypeStruct((N, D), jnp.float32),
               jax.ShapeDtypeStruct((N, D), jnp.float32)],
)


def _tc4_body(q_ref, hs2_ref, dis_ref, b2_ref, wc_ref, bc_ref,
              h2_ref, cls_ref):
    dis = dis_ref[...]
    agg = q_ref[0] + q_ref[1] + hs2_ref[...]
    h2 = jnp.maximum(dis * agg + b2_ref[...], 0.0)
    h2_ref[...] = h2
    cls_ref[...] = jnp.dot(h2, wc_ref[...],
                           preferred_element_type=jnp.float32) + bc_ref[...]


_tc4_call = pl.pallas_call(
    _tc4_body,
    grid=(GRID,),
    in_specs=[pl.BlockSpec((NC, BN, D), lambda i: (0, i, 0)),
              pl.BlockSpec((BN, D), lambda i: (i, 0)),
              pl.BlockSpec((BN, 1), lambda i: (i, 0)),
              pl.BlockSpec((1, D), lambda i: (0, 0)),
              pl.BlockSpec((D, C), lambda i: (0, 0)),
              pl.BlockSpec((1, C), lambda i: (0, 0))],
    out_specs=[pl.BlockSpec((BN, D), lambda i: (i, 0)),
               pl.BlockSpec((BN, C), lambda i: (i, 0))],
    out_shape=[jax.ShapeDtypeStruct((N, D), jnp.float32),
               jax.ShapeDtypeStruct((N, C), jnp.float32)],
)


def kernel(x, edge_index, W1, b1, W2, b2, Wc, bc):
    ei = edge_index.reshape(2, NW, EPW)
    src_p = jnp.concatenate(
        [ei[0], jnp.zeros((NW, EPWP - EPW), jnp.int32)], axis=1
    ).reshape(NW, NCH, CHUNK)
    dst_p = jnp.concatenate(
        [ei[1], jnp.full((NW, EPWP - EPW), NPAD - 1, jnp.int32)], axis=1
    ).reshape(NW, NCH, CHUNK)
    degs = _deg_kernel(dst_p)
    h1p = _h1p_call(x, W1)
    hs1, dis = _tc2_call(h1p, degs)

    p = _agg_kernel(hs1, src_p, dst_p)
    h1, hs2 = _tc3_call(p, hs1, dis, b1.reshape(1, D), W2)

    q = _agg_kernel(hs2, src_p, dst_p)
    h2, cls = _tc4_call(q, hs2, dis, b2.reshape(1, D), Wc, bc.reshape(1, C))

    feat_cat = jnp.concatenate([h1, h2, cls], axis=1)
    return (cls, feat_cat)


# speedup vs baseline: 2.9564x; 2.9564x over previous
import functools

import jax
import jax.numpy as jnp
from jax import lax
from jax.experimental import pallas as pl
from jax.experimental.pallas import tpu as pltpu
from jax.experimental.pallas import tpu_sc as plsc

N = 10000
NPAD = 10112
E = 320000
D = 128
C = 16

NC = 2
NS = 16
NW = NC * NS
EPW = E // NW
CHUNK = 128
NFULL = EPW // CHUNK
REM = EPW - NFULL * CHUNK
RPT = NPAD // NS

_mesh = plsc.VectorSubcoreMesh(core_axis_name="c", subcore_axis_name="s")


@functools.partial(
    pl.kernel,
    out_type=jax.ShapeDtypeStruct((NC, NPAD, 16), jnp.float32),
    mesh=_mesh,
    scratch_types=[
        pltpu.VMEM_SHARED((NPAD, 16), jnp.float32),
        pltpu.VMEM((CHUNK,), jnp.int32),
        pltpu.VMEM((REM,), jnp.int32),
        pltpu.VMEM((CHUNK, 16), jnp.float32),
        pltpu.VMEM((RPT, 16), jnp.float32),
    ],
)
def _deg_kernel(dst_hbm, out_hbm, deg_sh, didx, didx_r, ones_v, stage):
    cid = lax.axis_index("c")
    sid = lax.axis_index("s")
    wid = sid * NC + cid

    def fill(i, _):
        ones_v[i, pl.ds(0, 16)] = jnp.full((16,), 1.0, jnp.float32)
        return 0
    lax.fori_loop(0, CHUNK, fill, 0)

    def zfill(i, _):
        stage[i, pl.ds(0, 16)] = jnp.zeros((16,), jnp.float32)
        return 0
    lax.fori_loop(0, RPT, zfill, 0)

    r0 = sid * RPT
    pltpu.sync_copy(stage, deg_sh.at[pl.ds(r0, RPT)])
    plsc.subcore_barrier()

    base = wid * EPW

    def body(c, _):
        pltpu.sync_copy(dst_hbm.at[pl.ds(base + c * CHUNK, CHUNK)], didx)
        pltpu.sync_copy(ones_v, deg_sh.at[didx], add=True)
        return 0
    lax.fori_loop(0, NFULL, body, 0)
    pltpu.sync_copy(dst_hbm.at[pl.ds(base + NFULL * CHUNK, REM)], didx_r)
    pltpu.sync_copy(ones_v.at[pl.ds(0, REM)], deg_sh.at[didx_r], add=True)

    plsc.subcore_barrier()
    pltpu.sync_copy(deg_sh.at[pl.ds(r0, RPT)], stage)
    pltpu.sync_copy(stage, out_hbm.at[cid, pl.ds(r0, RPT)])


@functools.partial(
    pl.kernel,
    out_type=jax.ShapeDtypeStruct((NC, NPAD, D), jnp.float32),
    mesh=_mesh,
    scratch_types=[
        pltpu.VMEM_SHARED((NPAD, D), jnp.float32),
        pltpu.VMEM((CHUNK,), jnp.int32),
        pltpu.VMEM((CHUNK,), jnp.int32),
        pltpu.VMEM((CHUNK,), jnp.int32),
        pltpu.VMEM((CHUNK,), jnp.int32),
        pltpu.VMEM((CHUNK,), jnp.int32),
        pltpu.VMEM((CHUNK,), jnp.int32),
        pltpu.VMEM((CHUNK, D), jnp.float32),
        pltpu.VMEM((CHUNK, D), jnp.float32),
        pltpu.VMEM((CHUNK, D), jnp.float32),
        pltpu.VMEM((REM,), jnp.int32),
        pltpu.VMEM((REM,), jnp.int32),
        pltpu.SemaphoreType.DMA,
        pltpu.SemaphoreType.DMA,
        pltpu.SemaphoreType.DMA,
        pltpu.SemaphoreType.DMA,
        pltpu.SemaphoreType.DMA,
        pltpu.SemaphoreType.DMA,
    ],
)
def _agg_kernel(hs_hbm, src_hbm, dst_hbm, out_hbm, acc_sh,
                six0, six1, six2, dix0, dix1, dix2, rb0, rb1, rb2,
                sixr, dixr, gs0, gs1, gs2, ss0, ss1, ss2):
    cid = lax.axis_index("c")
    sid = lax.axis_index("s")
    wid = sid * NC + cid
    sixs = (six0, six1, six2)
    dixs = (dix0, dix1, dix2)
    rows = (rb0, rb1, rb2)
    gsems = (gs0, gs1, gs2)
    ssems = (ss0, ss1, ss2)

    def zfill(i, _):
        def zf2(j, _):
            rb0[i, pl.ds(j * 16, 16)] = jnp.zeros((16,), jnp.float32)
            return 0
        lax.fori_loop(0, D // 16, zf2, 0)
        return 0
    lax.fori_loop(0, CHUNK, zfill, 0)

    r0 = sid * RPT
    for k in range(4):
        pltpu.sync_copy(rb0, acc_sh.at[pl.ds(r0 + k * CHUNK, CHUNK)])
    pltpu.sync_copy(rb0.at[pl.ds(0, RPT - 4 * CHUNK)],
                    acc_sh.at[pl.ds(r0 + 4 * CHUNK, RPT - 4 * CHUNK)])
    plsc.subcore_barrier()

    base = wid * EPW

    for b in range(2):
        pltpu.sync_copy(src_hbm.at[pl.ds(base + b * CHUNK, CHUNK)], sixs[b])
        pltpu.sync_copy(dst_hbm.at[pl.ds(base + b * CHUNK, CHUNK)], dixs[b])
        pltpu.async_copy(hs_hbm.at[sixs[b]], rows[b], gsems[b])

    def visit(c, b, first, tail):
        pltpu.make_async_copy(hs_hbm.at[sixs[b]], rows[b], gsems[b]).wait()
        pltpu.async_copy(rows[b], acc_sh.at[dixs[b]], ssems[b], add=True)
        b2 = (b + 2) % 3
        if not first:
            pltpu.make_async_copy(rows[b2], acc_sh.at[dixs[b2]],
                                  ssems[b2]).wait()
        if not tail:
            off = base + (c + 2) * CHUNK
            pltpu.sync_copy(src_hbm.at[pl.ds(off, CHUNK)], sixs[b2])
            pltpu.sync_copy(dst_hbm.at[pl.ds(off, CHUNK)], dixs[b2])
            pltpu.async_copy(hs_hbm.at[sixs[b2]], rows[b2], gsems[b2])

    def body(g, _):
        for b3 in range(3):
            c = 3 * g + b3

            @pl.when(jnp.logical_and(c > 0, c < NFULL - 2))
            def _():
                visit(c, b3, False, False)
        return 0

    visit(0, 0, True, False)
    lax.fori_loop(0, NFULL // 3, body, 0)
    visit(NFULL - 2, (NFULL - 2) % 3, False, True)
    visit(NFULL - 1, (NFULL - 1) % 3, False, True)

    bl = (NFULL - 1) % 3
    pltpu.make_async_copy(rows[bl], acc_sh.at[dixs[bl]], ssems[bl]).wait()

    offr = base + NFULL * CHUNK
    pltpu.sync_copy(src_hbm.at[pl.ds(offr, REM)], sixr)
    pltpu.sync_copy(dst_hbm.at[pl.ds(offr, REM)], dixr)
    pltpu.async_copy(hs_hbm.at[sixr], rb0.at[pl.ds(0, REM)], gs0).wait()
    pltpu.sync_copy(rb0.at[pl.ds(0, REM)], acc_sh.at[dixr], add=True)

    plsc.subcore_barrier()
    for k in range(4):
        pltpu.sync_copy(acc_sh.at[pl.ds(r0 + k * CHUNK, CHUNK)], rb0)
        pltpu.sync_copy(rb0, out_hbm.at[cid, pl.ds(r0 + k * CHUNK, CHUNK)])
    pltpu.sync_copy(acc_sh.at[pl.ds(r0 + 4 * CHUNK, RPT - 4 * CHUNK)],
                    rb0.at[pl.ds(0, RPT - 4 * CHUNK)])
    pltpu.sync_copy(rb0.at[pl.ds(0, RPT - 4 * CHUNK)],
                    out_hbm.at[cid, pl.ds(r0 + 4 * CHUNK, RPT - 4 * CHUNK)])


BN = 1024
GRID = (N + BN - 1) // BN


def _split_body(ei_ref, src_ref, dst_ref):
    src_ref[...] = ei_ref[0]
    dst_ref[...] = ei_ref[1]


_split_call = pl.pallas_call(
    _split_body,
    out_shape=[jax.ShapeDtypeStruct((E,), jnp.int32),
               jax.ShapeDtypeStruct((E,), jnp.int32)],
)


def _tc1_body(x_ref, w1_ref, degs_ref, hs1_ref, dis_ref):
    deg = degs_ref[0, :, 0:1] + degs_ref[1, :, 0:1] + 1.0
    dis = lax.rsqrt(deg)
    dis_ref[...] = dis
    h1p = jnp.dot(x_ref[...], w1_ref[...], preferred_element_type=jnp.float32)
    hs1_ref[...] = h1p * dis


_tc1_call = pl.pallas_call(
    _tc1_body,
    grid=(GRID,),
    in_specs=[pl.BlockSpec((BN, D), lambda i: (i, 0)),
              pl.BlockSpec((D, D), lambda i: (0, 0)),
              pl.BlockSpec((NC, BN, 16), lambda i: (0, i, 0))],
    out_specs=[pl.BlockSpec((BN, D), lambda i: (i, 0)),
               pl.BlockSpec((BN, 1), lambda i: (i, 0))],
    out_shape=[jax.ShapeDtypeStruct((N, D), jnp.float32),
               jax.ShapeDtypeStruct((N, 1), jnp.float32)],
)


def _tc3_body(p_ref, hs1_ref, dis_ref, b1_ref, w2_ref, h1_ref, hs2_ref):
    dis = dis_ref[...]
    agg = p_ref[0] + p_ref[1] + hs1_ref[...]
    h1 = jnp.maximum(dis * agg + b1_ref[...], 0.0)
    h1_ref[...] = h1
    hs2_ref[...] = jnp.dot(h1, w2_ref[...],
                           preferred_element_type=jnp.float32) * dis


_tc3_call = pl.pallas_call(
    _tc3_body,
    grid=(GRID,),
    in_specs=[pl.BlockSpec((NC, BN, D), lambda i: (0, i, 0)),
              pl.BlockSpec((BN, D), lambda i: (i, 0)),
              pl.BlockSpec((BN, 1), lambda i: (i, 0)),
              pl.BlockSpec((1, D), lambda i: (0, 0)),
              pl.BlockSpec((D, D), lambda i: (0, 0))],
    out_specs=[pl.BlockSpec((BN, D), lambda i: (i, 0)),
               pl.BlockSpec((BN, D), lambda i: (i, 0))],
    out_shape=[jax.ShapeDtypeStruct((N, D), jnp.float32),
               jax.ShapeDtypeStruct((N, D), jnp.float32)],
)


def _tc4_body(q_ref, hs2_ref, h1_ref, dis_ref, b2_ref, wc_ref, bc_ref,
              cls_ref, feat_ref):
    dis = dis_ref[...]
    agg = q_ref[0] + q_ref[1] + hs2_ref[...]
    h2 = jnp.maximum(dis * agg + b2_ref[...], 0.0)
    cls = jnp.dot(h2, wc_ref[...], preferred_element_type=jnp.float32) \
        + bc_ref[...]
    cls_ref[...] = cls
    feat_ref[...] = jnp.concatenate([h1_ref[...], h2, cls], axis=1)


_tc4_call = pl.pallas_call(
    _tc4_body,
    grid=(GRID,),
    in_specs=[pl.BlockSpec((NC, BN, D), lambda i: (0, i, 0)),
              pl.BlockSpec((BN, D), lambda i: (i, 0)),
              pl.BlockSpec((BN, D), lambda i: (i, 0)),
              pl.BlockSpec((BN, 1), lambda i: (i, 0)),
              pl.BlockSpec((1, D), lambda i: (0, 0)),
              pl.BlockSpec((D, C), lambda i: (0, 0)),
              pl.BlockSpec((1, C), lambda i: (0, 0))],
    out_specs=[pl.BlockSpec((BN, C), lambda i: (i, 0)),
               pl.BlockSpec((BN, 2 * D + C), lambda i: (i, 0))],
    out_shape=[jax.ShapeDtypeStruct((N, C), jnp.float32),
               jax.ShapeDtypeStruct((N, 2 * D + C), jnp.float32)],
)


def kernel(x, edge_index, W1, b1, W2, b2, Wc, bc):
    src_e, dst_e = _split_call(edge_index)

    degs = _deg_kernel(dst_e)
    hs1, dis = _tc1_call(x, W1, degs)

    p = _agg_kernel(hs1, src_e, dst_e)
    h1, hs2 = _tc3_call(p, hs1, dis, b1.reshape(1, D), W2)

    q = _agg_kernel(hs2, src_e, dst_e)
    cls, feat_cat = _tc4_call(q, hs2, h1, dis, b2.reshape(1, D), Wc,
                              bc.reshape(1, C))

    return (cls, feat_cat)


# speedup vs baseline: 3.2525x; 1.1002x over previous
import functools

import jax
import jax.numpy as jnp
from jax import lax
from jax.experimental import pallas as pl
from jax.experimental.pallas import tpu as pltpu
from jax.experimental.pallas import tpu_sc as plsc

N = 10000
NPAD = 10112
E = 320000
D = 128
C = 16

NC = 2
NS = 16
NW = NC * NS
EPW = E // NW
CHUNK = 128
NFULL = EPW // CHUNK
REM = EPW - NFULL * CHUNK
RPT = NPAD // NS

_mesh = plsc.VectorSubcoreMesh(core_axis_name="c", subcore_axis_name="s")


@functools.partial(
    pl.kernel,
    out_type=jax.ShapeDtypeStruct((NC, NPAD, 16), jnp.float32),
    mesh=_mesh,
    scratch_types=[
        pltpu.VMEM_SHARED((NPAD, 16), jnp.float32),
        pltpu.VMEM((CHUNK,), jnp.int32),
        pltpu.VMEM((CHUNK,), jnp.int32),
        pltpu.VMEM((CHUNK,), jnp.int32),
        pltpu.VMEM((REM,), jnp.int32),
        pltpu.VMEM((CHUNK, 16), jnp.float32),
        pltpu.VMEM((RPT, 16), jnp.float32),
        pltpu.SemaphoreType.DMA,
        pltpu.SemaphoreType.DMA,
        pltpu.SemaphoreType.DMA,
        pltpu.SemaphoreType.DMA,
    ],
)
def _deg_kernel(dst_hbm, out_hbm, deg_sh, di0, di1, di2, didx_r,
                ones_v, stage, ss0, ss1, ss2, isem):
    cid = lax.axis_index("c")
    sid = lax.axis_index("s")
    wid = sid * NC + cid
    didxs = (di0, di1, di2)
    ssems = (ss0, ss1, ss2)

    def fill(i, _):
        ones_v[i, pl.ds(0, 16)] = jnp.full((16,), 1.0, jnp.float32)
        return 0
    lax.fori_loop(0, CHUNK, fill, 0)

    def zfill(i, _):
        stage[i, pl.ds(0, 16)] = jnp.zeros((16,), jnp.float32)
        return 0
    lax.fori_loop(0, RPT, zfill, 0)

    r0 = sid * RPT
    pltpu.sync_copy(stage, deg_sh.at[pl.ds(r0, RPT)])
    plsc.subcore_barrier()

    base = wid * EPW
    for b in range(2):
        pltpu.sync_copy(dst_hbm.at[pl.ds(base + b * CHUNK, CHUNK)], didxs[b])

    def visit(c, b, first, tail):
        pltpu.async_copy(ones_v, deg_sh.at[didxs[b]], ssems[b], add=True)
        b2 = (b + 2) % 3
        if not first:
            pltpu.make_async_copy(ones_v, deg_sh.at[didxs[b2]],
                                  ssems[b2]).wait()
        if not tail:
            off = base + (c + 2) * CHUNK
            pltpu.async_copy(dst_hbm.at[pl.ds(off, CHUNK)], didxs[b2],
                             isem).wait()

    def body(g, _):
        for b3 in range(3):
            c = 3 * g + b3

            @pl.when(jnp.logical_and(c > 0, c < NFULL - 2))
            def _():
                visit(c, b3, False, False)
        return 0

    visit(0, 0, True, False)
    lax.fori_loop(0, NFULL // 3, body, 0)
    visit(NFULL - 2, (NFULL - 2) % 3, False, True)
    visit(NFULL - 1, (NFULL - 1) % 3, False, True)
    bl = (NFULL - 1) % 3
    pltpu.make_async_copy(ones_v, deg_sh.at[didxs[bl]], ssems[bl]).wait()

    pltpu.sync_copy(dst_hbm.at[pl.ds(base + NFULL * CHUNK, REM)], didx_r)
    pltpu.sync_copy(ones_v.at[pl.ds(0, REM)], deg_sh.at[didx_r], add=True)

    plsc.subcore_barrier()
    pltpu.sync_copy(deg_sh.at[pl.ds(r0, RPT)], stage)
    pltpu.sync_copy(stage, out_hbm.at[cid, pl.ds(r0, RPT)])


@functools.partial(
    pl.kernel,
    out_type=jax.ShapeDtypeStruct((NC, NPAD, D), jnp.float32),
    mesh=_mesh,
    scratch_types=[
        pltpu.VMEM_SHARED((NPAD, D), jnp.float32),
        pltpu.VMEM((CHUNK,), jnp.int32),
        pltpu.VMEM((CHUNK,), jnp.int32),
        pltpu.VMEM((CHUNK,), jnp.int32),
        pltpu.VMEM((CHUNK,), jnp.int32),
        pltpu.VMEM((CHUNK,), jnp.int32),
        pltpu.VMEM((CHUNK,), jnp.int32),
        pltpu.VMEM((CHUNK, D), jnp.float32),
        pltpu.VMEM((CHUNK, D), jnp.float32),
        pltpu.VMEM((CHUNK, D), jnp.float32),
        pltpu.VMEM((REM,), jnp.int32),
        pltpu.VMEM((REM,), jnp.int32),
        pltpu.SemaphoreType.DMA,
        pltpu.SemaphoreType.DMA,
        pltpu.SemaphoreType.DMA,
        pltpu.SemaphoreType.DMA,
        pltpu.SemaphoreType.DMA,
        pltpu.SemaphoreType.DMA,
        pltpu.SemaphoreType.DMA,
        pltpu.SemaphoreType.DMA,
    ],
)
def _agg_kernel(hs_hbm, src_hbm, dst_hbm, out_hbm, acc_sh,
                six0, six1, six2, dix0, dix1, dix2, rb0, rb1, rb2,
                sixr, dixr, gs0, gs1, gs2, ss0, ss1, ss2, is0, is1):
    cid = lax.axis_index("c")
    sid = lax.axis_index("s")
    wid = sid * NC + cid
    sixs = (six0, six1, six2)
    dixs = (dix0, dix1, dix2)
    rows = (rb0, rb1, rb2)
    gsems = (gs0, gs1, gs2)
    ssems = (ss0, ss1, ss2)

    def zfill(i, _):
        def zf2(j, _):
            rb0[i, pl.ds(j * 16, 16)] = jnp.zeros((16,), jnp.float32)
            return 0
        lax.fori_loop(0, D // 16, zf2, 0)
        return 0
    lax.fori_loop(0, CHUNK, zfill, 0)

    r0 = sid * RPT
    for k in range(4):
        pltpu.sync_copy(rb0, acc_sh.at[pl.ds(r0 + k * CHUNK, CHUNK)])
    pltpu.sync_copy(rb0.at[pl.ds(0, RPT - 4 * CHUNK)],
                    acc_sh.at[pl.ds(r0 + 4 * CHUNK, RPT - 4 * CHUNK)])
    plsc.subcore_barrier()

    base = wid * EPW

    for b in range(2):
        pltpu.sync_copy(src_hbm.at[pl.ds(base + b * CHUNK, CHUNK)], sixs[b])
        pltpu.sync_copy(dst_hbm.at[pl.ds(base + b * CHUNK, CHUNK)], dixs[b])
        pltpu.async_copy(hs_hbm.at[sixs[b]], rows[b], gsems[b])

    def visit(c, b, first, tail):
        pltpu.make_async_copy(hs_hbm.at[sixs[b]], rows[b], gsems[b]).wait()
        pltpu.async_copy(rows[b], acc_sh.at[dixs[b]], ssems[b], add=True)
        b2 = (b + 2) % 3
        d1 = None
        if not tail:
            off = base + (c + 2) * CHUNK
            d1 = pltpu.async_copy(src_hbm.at[pl.ds(off, CHUNK)], sixs[b2],
                                  is0)
        if not first:
            pltpu.make_async_copy(rows[b2], acc_sh.at[dixs[b2]],
                                  ssems[b2]).wait()
        if not tail:
            d2 = pltpu.async_copy(dst_hbm.at[pl.ds(off, CHUNK)], dixs[b2],
                                  is1)
            d1.wait()
            d2.wait()
            pltpu.async_copy(hs_hbm.at[sixs[b2]], rows[b2], gsems[b2])

    def body(g, _):
        for b3 in range(3):
            c = 3 * g + b3

            @pl.when(jnp.logical_and(c > 0, c < NFULL - 2))
            def _():
                visit(c, b3, False, False)
        return 0

    visit(0, 0, True, False)
    lax.fori_loop(0, NFULL // 3, body, 0)
    visit(NFULL - 2, (NFULL - 2) % 3, False, True)
    visit(NFULL - 1, (NFULL - 1) % 3, False, True)

    bl = (NFULL - 1) % 3
    pltpu.make_async_copy(rows[bl], acc_sh.at[dixs[bl]], ssems[bl]).wait()

    offr = base + NFULL * CHUNK
    pltpu.sync_copy(src_hbm.at[pl.ds(offr, REM)], sixr)
    pltpu.sync_copy(dst_hbm.at[pl.ds(offr, REM)], dixr)
    pltpu.async_copy(hs_hbm.at[sixr], rb0.at[pl.ds(0, REM)], gs0).wait()
    pltpu.sync_copy(rb0.at[pl.ds(0, REM)], acc_sh.at[dixr], add=True)

    plsc.subcore_barrier()
    for k in range(4):
        pltpu.sync_copy(acc_sh.at[pl.ds(r0 + k * CHUNK, CHUNK)], rb0)
        pltpu.sync_copy(rb0, out_hbm.at[cid, pl.ds(r0 + k * CHUNK, CHUNK)])
    pltpu.sync_copy(acc_sh.at[pl.ds(r0 + 4 * CHUNK, RPT - 4 * CHUNK)],
                    rb0.at[pl.ds(0, RPT - 4 * CHUNK)])
    pltpu.sync_copy(rb0.at[pl.ds(0, RPT - 4 * CHUNK)],
                    out_hbm.at[cid, pl.ds(r0 + 4 * CHUNK, RPT - 4 * CHUNK)])


BN = 1024
GRID = (N + BN - 1) // BN


def _split_body(ei_ref, src_ref, dst_ref):
    src_ref[...] = ei_ref[0]
    dst_ref[...] = ei_ref[1]


_split_call = pl.pallas_call(
    _split_body,
    out_shape=[jax.ShapeDtypeStruct((E,), jnp.int32),
               jax.ShapeDtypeStruct((E,), jnp.int32)],
)


def _tc1_body(x_ref, w1_ref, degs_ref, hs1_ref, dis_ref):
    deg = degs_ref[0, :, 0:1] + degs_ref[1, :, 0:1] + 1.0
    dis = lax.rsqrt(deg)
    dis_ref[...] = dis
    h1p = jnp.dot(x_ref[...], w1_ref[...], preferred_element_type=jnp.float32)
    hs1_ref[...] = h1p * dis


_tc1_call = pl.pallas_call(
    _tc1_body,
    grid=(GRID,),
    in_specs=[pl.BlockSpec((BN, D), lambda i: (i, 0)),
              pl.BlockSpec((D, D), lambda i: (0, 0)),
              pl.BlockSpec((NC, BN, 16), lambda i: (0, i, 0))],
    out_specs=[pl.BlockSpec((BN, D), lambda i: (i, 0)),
               pl.BlockSpec((BN, 1), lambda i: (i, 0))],
    out_shape=[jax.ShapeDtypeStruct((N, D), jnp.float32),
               jax.ShapeDtypeStruct((N, 1), jnp.float32)],
)


def _tc3_body(p_ref, hs1_ref, dis_ref, b1_ref, w2_ref, h1_ref, hs2_ref):
    dis = dis_ref[...]
    agg = p_ref[0] + p_ref[1] + hs1_ref[...]
    h1 = jnp.maximum(dis * agg + b1_ref[...], 0.0)
    h1_ref[...] = h1
    hs2_ref[...] = jnp.dot(h1, w2_ref[...],
                           preferred_element_type=jnp.float32) * dis


_tc3_call = pl.pallas_call(
    _tc3_body,
    grid=(GRID,),
    in_specs=[pl.BlockSpec((NC, BN, D), lambda i: (0, i, 0)),
              pl.BlockSpec((BN, D), lambda i: (i, 0)),
              pl.BlockSpec((BN, 1), lambda i: (i, 0)),
              pl.BlockSpec((1, D), lambda i: (0, 0)),
              pl.BlockSpec((D, D), lambda i: (0, 0))],
    out_specs=[pl.BlockSpec((BN, D), lambda i: (i, 0)),
               pl.BlockSpec((BN, D), lambda i: (i, 0))],
    out_shape=[jax.ShapeDtypeStruct((N, D), jnp.float32),
               jax.ShapeDtypeStruct((N, D), jnp.float32)],
)


def _tc4_body(q_ref, hs2_ref, h1_ref, dis_ref, b2_ref, wc_ref, bc_ref,
              cls_ref, feat_ref):
    dis = dis_ref[...]
    agg = q_ref[0] + q_ref[1] + hs2_ref[...]
    h2 = jnp.maximum(dis * agg + b2_ref[...], 0.0)
    cls = jnp.dot(h2, wc_ref[...], preferred_element_type=jnp.float32) \
        + bc_ref[...]
    cls_ref[...] = cls
    feat_ref[...] = jnp.concatenate([h1_ref[...], h2, cls], axis=1)


_tc4_call = pl.pallas_call(
    _tc4_body,
    grid=(GRID,),
    in_specs=[pl.BlockSpec((NC, BN, D), lambda i: (0, i, 0)),
              pl.BlockSpec((BN, D), lambda i: (i, 0)),
              pl.BlockSpec((BN, D), lambda i: (i, 0)),
              pl.BlockSpec((BN, 1), lambda i: (i, 0)),
              pl.BlockSpec((1, D), lambda i: (0, 0)),
              pl.BlockSpec((D, C), lambda i: (0, 0)),
              pl.BlockSpec((1, C), lambda i: (0, 0))],
    out_specs=[pl.BlockSpec((BN, C), lambda i: (i, 0)),
               pl.BlockSpec((BN, 2 * D + C), lambda i: (i, 0))],
    out_shape=[jax.ShapeDtypeStruct((N, C), jnp.float32),
               jax.ShapeDtypeStruct((N, 2 * D + C), jnp.float32)],
)


def kernel(x, edge_index, W1, b1, W2, b2, Wc, bc):
    src_e, dst_e = _split_call(edge_index)

    degs = _deg_kernel(dst_e)
    hs1, dis = _tc1_call(x, W1, degs)

    p = _agg_kernel(hs1, src_e, dst_e)
    h1, hs2 = _tc3_call(p, hs1, dis, b1.reshape(1, D), W2)

    q = _agg_kernel(hs2, src_e, dst_e)
    cls, feat_cat = _tc4_call(q, hs2, h1, dis, b2.reshape(1, D), Wc,
                              bc.reshape(1, C))

    return (cls, feat_cat)
